# Initial kernel scaffold; baseline (speedup 1.0000x reference)
import jax
import jax.numpy as jnp
from jax.experimental import pallas as pl


def kernel(xyz, f):
    raise NotImplementedError("write your pallas kernel here")


# speedup vs baseline: 20.6904x; 20.6904x over previous
import functools

import jax
import jax.numpy as jnp
from jax import lax
from jax.experimental import pallas as pl
from jax.experimental.pallas import tpu as pltpu
from jax.experimental.pallas import tpu_sc as plsc

B = 8
N = 8192
S = 512
D = 256
GROUP = 4
SHARD = N // GROUP
CHUNKS = SHARD // 16
ROWS = S // GROUP
BIGI = 0x7FFFFFFF


def _fps_body(xyzp, f, xyz_flat_out, f_out,
              x_ref, y_ref, z_ref, dist_ref, pub, cons, idxbuf, xyzflat,
              idxg, fbuf, sh, sem):
    c = lax.axis_index("c")
    s_id = lax.axis_index("s")
    b = c * 4 + s_id // 4
    m = s_id % 4
    g0 = (s_id // 4) * 4
    base = m * SHARD
    iota = lax.iota(jnp.int32, 16)

    pltpu.sync_copy(xyzp.at[pl.ds((b * 3 + 0) * N + base, SHARD)], x_ref)
    pltpu.sync_copy(xyzp.at[pl.ds((b * 3 + 1) * N + base, SHARD)], y_ref)
    pltpu.sync_copy(xyzp.at[pl.ds((b * 3 + 2) * N + base, SHARD)], z_ref)

    big = jnp.full((16,), 1e10, jnp.float32)

    def init_body(i, carry):
        dist_ref[pl.ds(i * 16, 16)] = big
        return carry

    lax.fori_loop(0, CHUNKS, init_body, 0)

    def publish(wbuf, valv, idxv_i32, cxv, cyv, czv):
        pub[pl.ds(0, 16)] = valv
        pub[pl.ds(16, 16)] = plsc.bitcast(idxv_i32, jnp.float32)
        pub[pl.ds(32, 16)] = cxv
        pub[pl.ds(48, 16)] = cyv
        pub[pl.ds(64, 16)] = czv
        pltpu.sync_copy(pub, sh.at[pl.ds(wbuf * 1280 + s_id * 80, 80)])

    def lane0_splat(ref):
        v = ref[pl.ds(0, 16)]
        top = jnp.max(jnp.where(iota == 0, v, jnp.float32(-1.0)))
        return jnp.full((16,), top, jnp.float32)

    val0 = jnp.where(m == 0, jnp.float32(1.0), jnp.float32(-1.0))
    publish(0, jnp.full((16,), val0, jnp.float32), jnp.zeros((16,), jnp.int32),
            lane0_splat(x_ref), lane0_splat(y_ref), lane0_splat(z_ref))
    plsc.subcore_barrier()

    def step(s_step, rbuf, wbuf):
        pltpu.sync_copy(sh.at[pl.ds(rbuf * 1280 + g0 * 80, GROUP * 80)], cons)
        v = cons[pl.ds(0, 16)]
        pidx = cons[pl.ds(16, 16)]
        px = cons[pl.ds(32, 16)]
        py = cons[pl.ds(48, 16)]
        pz = cons[pl.ds(64, 16)]
        for r in range(1, GROUP):
            o = r * 80
            vr = cons[pl.ds(o, 16)]
            mk = vr > v
            v = jnp.where(mk, vr, v)
            pidx = jnp.where(mk, cons[pl.ds(o + 16, 16)], pidx)
            px = jnp.where(mk, cons[pl.ds(o + 32, 16)], px)
            py = jnp.where(mk, cons[pl.ds(o + 48, 16)], py)
            pz = jnp.where(mk, cons[pl.ds(o + 64, 16)], pz)
        gidxv = plsc.bitcast(pidx, jnp.int32)

        posv = 3 * s_step + iota
        valrec = jnp.where(iota == 0, px, jnp.where(iota == 1, py, pz))
        plsc.store_scatter(xyzflat, [posv], valrec, mask=iota < 3)
        plsc.store_scatter(idxbuf, [jnp.full((16,), s_step, jnp.int32)],
                           gidxv, mask=iota == 0)

        def chunk_body(i, carry):
            rmax, ridx = carry
            sl = pl.ds(i * 16, 16)
            xv = x_ref[sl]
            yv = y_ref[sl]
            zv = z_ref[sl]
            dv = dist_ref[sl]
            dx = xv - px
            dy = yv - py
            dz = zv - pz
            d = (dx * dx + dy * dy) + dz * dz
            dn = jnp.minimum(dv, d)
            dist_ref[sl] = dn
            mk = dn > rmax
            rmax = jnp.where(mk, dn, rmax)
            ridx = jnp.where(mk, base + i * 16 + iota, ridx)
            return rmax, ridx

        rmax, ridx = lax.fori_loop(
            0, CHUNKS, chunk_body,
            (jnp.full((16,), -1.0, jnp.float32), jnp.zeros((16,), jnp.int32)))

        gmax = jnp.max(rmax)
        gmaxv = jnp.full((16,), gmax, jnp.float32)
        cand = jnp.where(rmax == gmaxv, ridx, BIGI)
        gidx = jnp.min(cand)
        gidxv2 = jnp.full((16,), gidx, jnp.int32)
        lidxv = gidxv2 - base
        publish(wbuf, gmaxv, gidxv2,
                plsc.load_gather(x_ref, [lidxv]),
                plsc.load_gather(y_ref, [lidxv]),
                plsc.load_gather(z_ref, [lidxv]))
        plsc.subcore_barrier()

    def outer_body(i, carry):
        step(2 * i, 0, 1)
        step(2 * i + 1, 1, 0)
        return carry

    lax.fori_loop(0, S // 2, outer_body, 0)

    def idx_copy(j, carry):
        idxg[pl.ds(j * 16, 16)] = idxbuf[pl.ds(m * ROWS + j * 16, 16)] + b * N
        return carry

    lax.fori_loop(0, ROWS // 16, idx_copy, 0)
    pltpu.async_copy(f.at[idxg], fbuf, sem).wait()
    pltpu.sync_copy(fbuf, f_out.at[pl.ds(b * S + m * ROWS, ROWS)])

    @pl.when(m == 0)
    def _():
        pltpu.sync_copy(xyzflat, xyz_flat_out.at[pl.ds(b * S * 3, S * 3)])


@jax.jit
def kernel(xyz, f):
    xyzp = jnp.transpose(xyz, (0, 2, 1)).reshape(B * 3 * N)
    f2d = f.reshape(B * N, D)
    mesh = plsc.VectorSubcoreMesh(core_axis_name="c", subcore_axis_name="s")
    fps = pl.kernel(
        _fps_body,
        out_type=(
            jax.ShapeDtypeStruct((B * S * 3,), jnp.float32),
            jax.ShapeDtypeStruct((B * S, D), jnp.float32),
        ),
        mesh=mesh,
        compiler_params=pltpu.CompilerParams(needs_layout_passes=False),
        scratch_types=[
            pltpu.VMEM((SHARD,), jnp.float32),
            pltpu.VMEM((SHARD,), jnp.float32),
            pltpu.VMEM((SHARD,), jnp.float32),
            pltpu.VMEM((SHARD,), jnp.float32),
            pltpu.VMEM((80,), jnp.float32),
            pltpu.VMEM((GROUP * 80,), jnp.float32),
            pltpu.VMEM((S,), jnp.int32),
            pltpu.VMEM((S * 3,), jnp.float32),
            pltpu.VMEM((ROWS,), jnp.int32),
            pltpu.VMEM((ROWS, D), jnp.float32),
            pltpu.VMEM_SHARED((2 * 16 * 80,), jnp.float32),
            pltpu.SemaphoreType.DMA,
        ],
    )
    xyz_flat, f_sampled = fps(xyzp, f2d)
    return xyz_flat.reshape(B, S, 3), f_sampled.reshape(B, S, D)
